# baseline (speedup 1.0000x reference)
import functools

import jax
import jax.numpy as jnp
from jax import lax
from jax.experimental import pallas as pl
from jax.experimental.pallas import tpu as pltpu
from jax.experimental.pallas import tpu_sc as plsc

B = 8
N = 1024
K = 20
CP = 128
NP = 256
NEG = -3.0e38


def _topk_into(d_ref, idx_ref, lane, boff):
    for t in range(K):
        d = d_ref[...]
        am = jnp.argmax(d, axis=1).astype(jnp.int32)
        idx_ref[0, :, t] = am + boff
        d_ref[...] = jnp.where(lane == am[:, None], NEG, d)


def _dist_body(c, a_ref, idx_ref, d_ref):
    h = a_ref[0][:, :c]
    xx = jnp.sum(h * h, axis=1)
    hb = h.astype(jnp.bfloat16)
    g = lax.dot_general(hb, hb, (((1,), (1,)), ((), ())),
                        preferred_element_type=jnp.float32)
    d_ref[...] = (2.0 * g - xx[:, None]) - xx[None, :]
    lane = lax.broadcasted_iota(jnp.int32, (N, N), 1)
    _topk_into(d_ref, idx_ref, lane, pl.program_id(0) * N)


def _run_dist(h_pad, c):
    nb = h_pad.shape[0]
    return pl.pallas_call(
        functools.partial(_dist_body, c),
        grid=(nb,),
        in_specs=[pl.BlockSpec((1, N, h_pad.shape[2]), lambda b: (b, 0, 0))],
        out_specs=pl.BlockSpec((1, N, K), lambda b: (b, 0, 0)),
        out_shape=jax.ShapeDtypeStruct((nb, N, K), jnp.int32),
        scratch_shapes=[pltpu.VMEM((N, N), jnp.float32)],
    )(h_pad)


def _gather_sc(table_flat, idx_flat, co):
    NW = 32
    npts = table_flat.shape[0]
    cw = table_flat.shape[1]
    P = npts // NW
    CW = 128
    NCHUNK = (P * K) // CW
    mesh = plsc.VectorSubcoreMesh(core_axis_name="c", subcore_axis_name="s")

    @functools.partial(
        pl.kernel, mesh=mesh,
        out_type=jax.ShapeDtypeStruct((npts * K, co), jnp.float32),
        scratch_types=[
            pltpu.VMEM((P * K,), jnp.int32),
            pltpu.VMEM((2, CW, cw), jnp.float32),
            pltpu.SemaphoreType.DMA,
            pltpu.SemaphoreType.DMA,
            pltpu.SemaphoreType.DMA,
        ],
        compiler_params=pltpu.CompilerParams(use_tc_tiling_on_sc=False),
    )
    def gsc(tab_hbm, idx_hbm, out_hbm, idx_v, rows_v, gsem, osem0, osem1):
        wid = lax.axis_index("s") * 2 + lax.axis_index("c")
        base = wid * P * K
        pltpu.sync_copy(idx_hbm.at[pl.ds(base, P * K)], idx_v)
        osems = (osem0, osem1)

        def gather(c):
            return pltpu.async_copy(
                tab_hbm.at[idx_v.at[pl.ds(c * CW, CW)]],
                rows_v.at[c % 2], gsem)

        def put(c):
            src = rows_v.at[c % 2]
            if co < cw:
                src = src.at[:, pl.ds(0, co)]
            return pltpu.async_copy(
                src, out_hbm.at[pl.ds(base + c * CW, CW)], osems[c % 2])

        g = gather(0)
        prev_w = None
        for c in range(NCHUNK):
            g.wait()
            w = put(c)
            if prev_w is not None:
                prev_w.wait()
            if c + 1 < NCHUNK:
                g = gather(c + 1)
            prev_w = w
        prev_w.wait()

    return gsc(table_flat, idx_flat)


def _edge_body(c, cout, cpo, feats_ref, hprev_ref, waT_ref, wbT_ref, b_ref,
               out_ref):
    feat = feats_ref[0][:, :c]
    center = hprev_ref[0][:, :c]
    feat3 = feat.reshape(NP, K, c)
    ea = (feat3 - center[:, None, :]).astype(jnp.bfloat16)
    za = lax.dot_general(ea.reshape(NP * K, c), waT_ref[...],
                         (((1,), (0,)), ((), ())),
                         preferred_element_type=jnp.float32)
    cb = center.astype(jnp.bfloat16)
    zb = lax.dot_general(cb, wbT_ref[...], (((1,), (0,)), ((), ())),
                         preferred_element_type=jnp.float32)
    z = za.reshape(NP, K, cout) + zb[:, None, :]
    z = z + b_ref[...][None]
    z = jnp.where(z >= 0, z, 0.2 * z)
    hout = jnp.max(z, axis=1)
    if cpo > cout:
        hout = jnp.concatenate(
            [hout, jnp.zeros((NP, cpo - cout), jnp.float32)], axis=1)
    out_ref[0] = hout


def _run_edge(feats, h_prev, waT_bf, wbT_bf, bias, cpo):
    c = waT_bf.shape[0]
    cout = waT_bf.shape[1]
    nc = N // NP
    nb = feats.shape[0]
    full = lambda b, i: (0, 0)
    return pl.pallas_call(
        functools.partial(_edge_body, c, cout, cpo),
        grid=(nb, nc),
        in_specs=[
            pl.BlockSpec((1, NP * K, feats.shape[2]), lambda b, i: (b, i, 0)),
            pl.BlockSpec((1, NP, h_prev.shape[2]), lambda b, i: (b, i, 0)),
            pl.BlockSpec((c, cout), full),
            pl.BlockSpec((c, cout), full),
            pl.BlockSpec((1, cout), full),
        ],
        out_specs=pl.BlockSpec((1, NP, cpo), lambda b, i: (b, i, 0)),
        out_shape=jax.ShapeDtypeStruct((nb, N, cpo), jnp.float32),
    )(feats, h_prev, waT_bf, wbT_bf, bias.reshape(1, cout))


def _final_body(h1_ref, h2_ref, h3_ref, h4_ref,
                w1_ref, w2_ref, w3_ref, w4_ref, b_ref, out_ref):
    acc = lax.dot_general(h1_ref[0][:, :64].astype(jnp.bfloat16), w1_ref[...],
                          (((1,), (0,)), ((), ())),
                          preferred_element_type=jnp.float32)
    acc += lax.dot_general(h2_ref[0][:, :64].astype(jnp.bfloat16), w2_ref[...],
                           (((1,), (0,)), ((), ())),
                           preferred_element_type=jnp.float32)
    acc += lax.dot_general(h3_ref[0].astype(jnp.bfloat16), w3_ref[...],
                           (((1,), (0,)), ((), ())),
                           preferred_element_type=jnp.float32)
    acc += lax.dot_general(h4_ref[0].astype(jnp.bfloat16), w4_ref[...],
                           (((1,), (0,)), ((), ())),
                           preferred_element_type=jnp.float32)
    out_ref[0] = jnp.max(acc, axis=0)[None, :] + b_ref[...]


def _run_final(h1, h2, h3, h4, wf_parts, bf):
    fdim = bf.shape[0]
    nb = h1.shape[0]
    full = lambda b: (0, 0)
    bat = lambda b: (b, 0, 0)
    in_specs = [
        pl.BlockSpec((1, N, h1.shape[2]), bat),
        pl.BlockSpec((1, N, h2.shape[2]), bat),
        pl.BlockSpec((1, N, h3.shape[2]), bat),
        pl.BlockSpec((1, N, h4.shape[2]), bat),
    ] + [pl.BlockSpec(w.shape, full) for w in wf_parts] \
      + [pl.BlockSpec((1, fdim), full)]
    return pl.pallas_call(
        _final_body,
        grid=(nb,),
        in_specs=in_specs,
        out_specs=pl.BlockSpec((1, 1, fdim), lambda b: (b, 0, 0)),
        out_shape=jax.ShapeDtypeStruct((nb, 1, fdim), jnp.float32),
    )(h1, h2, h3, h4, *wf_parts, bf.reshape(1, fdim)).reshape(nb, fdim)


NGROUP = 2


def _group_chain(x, Ws, bs, wf_parts, bf):
    cins = (3, 64, 64, 128)
    cws = (8, 64, 64, 128)
    pads = (128, 128, 128, 256)
    nb = x.shape[0]
    h_pad = jnp.pad(x, ((0, 0), (0, 0), (0, CP - x.shape[2])))
    acts = []
    for i in range(4):
        c = cins[i]
        cw = cws[i]
        waT = jnp.transpose(Ws[i][:, :c]).astype(jnp.bfloat16)
        wbT = jnp.transpose(Ws[i][:, c:]).astype(jnp.bfloat16)
        idx = _run_dist(h_pad, c)
        feats = _gather_sc(h_pad[:, :, :cw].reshape(nb * N, cw),
                           idx.reshape(nb * N * K), cw)
        feats = feats.reshape(nb, N * K, cw)
        h_next = _run_edge(feats, h_pad, waT, wbT, bs[i], pads[i])
        acts.append(h_next)
        h_pad = h_next
    return _run_final(acts[0], acts[1], acts[2], acts[3], wf_parts, bf)


def kernel(x, W0, b0, W1, b1, W2, b2, W3, b3, Wf, bf):
    Ws = (W0, W1, W2, W3)
    bs = (b0, b1, b2, b3)
    wfT = jnp.transpose(Wf).astype(jnp.bfloat16)
    wf_parts = [wfT[0:64], wfT[64:128], wfT[128:256], wfT[256:512]]
    gb = B // NGROUP
    outs = [_group_chain(x[g * gb:(g + 1) * gb], Ws, bs, wf_parts, bf)
            for g in range(NGROUP)]
    return jnp.concatenate(outs, axis=0)


# speedup vs baseline: 1.1885x; 1.1885x over previous
import functools

import jax
import jax.numpy as jnp
from jax import lax
from jax.experimental import pallas as pl
from jax.experimental.pallas import tpu as pltpu
from jax.experimental.pallas import tpu_sc as plsc

B = 8
N = 1024
K = 20
CP = 128
NP = 256
NEG = -3.0e38


def _topk_into(d_ref, idx_ref, lane, boff):
    for t in range(K):
        d = d_ref[...]
        am = jnp.argmax(d, axis=1).astype(jnp.int32)
        idx_ref[0, :, t] = am + boff
        d_ref[...] = jnp.where(lane == am[:, None], NEG, d)


def _dist_body(c, a_ref, idx_ref, d_ref):
    h = a_ref[0][:, :c]
    xx = jnp.sum(h * h, axis=1)
    hb = h.astype(jnp.bfloat16)
    g = lax.dot_general(hb, hb, (((1,), (1,)), ((), ())),
                        preferred_element_type=jnp.float32)
    d_ref[...] = (2.0 * g - xx[:, None]) - xx[None, :]
    lane = lax.broadcasted_iota(jnp.int32, (N, N), 1)
    _topk_into(d_ref, idx_ref, lane, pl.program_id(0) * N)


def _run_dist(h_pad, c):
    nb = h_pad.shape[0]
    return pl.pallas_call(
        functools.partial(_dist_body, c),
        grid=(nb,),
        in_specs=[pl.BlockSpec((1, N, h_pad.shape[2]), lambda b: (b, 0, 0))],
        out_specs=pl.BlockSpec((1, N, K), lambda b: (b, 0, 0)),
        out_shape=jax.ShapeDtypeStruct((nb, N, K), jnp.int32),
        scratch_shapes=[pltpu.VMEM((N, N), jnp.float32)],
    )(h_pad)


def _gather_sc(table_flat, idx_flat, co):
    NW = 32
    npts = table_flat.shape[0]
    cw = table_flat.shape[1]
    P = npts // NW
    CW = 128
    NCHUNK = (P * K) // CW
    mesh = plsc.VectorSubcoreMesh(core_axis_name="c", subcore_axis_name="s")

    @functools.partial(
        pl.kernel, mesh=mesh,
        out_type=jax.ShapeDtypeStruct((npts * K, co), jnp.float32),
        scratch_types=[
            pltpu.VMEM((P * K,), jnp.int32),
            pltpu.VMEM((2, CW, cw), jnp.float32),
            pltpu.SemaphoreType.DMA,
            pltpu.SemaphoreType.DMA,
            pltpu.SemaphoreType.DMA,
        ],
    )
    def gsc(tab_hbm, idx_hbm, out_hbm, idx_v, rows_v, gsem, osem0, osem1):
        wid = lax.axis_index("s") * 2 + lax.axis_index("c")
        base = wid * P * K
        pltpu.sync_copy(idx_hbm.at[pl.ds(base, P * K)], idx_v)
        osems = (osem0, osem1)

        def gather(c):
            return pltpu.async_copy(
                tab_hbm.at[idx_v.at[pl.ds(c * CW, CW)]],
                rows_v.at[c % 2], gsem)

        def put(c):
            src = rows_v.at[c % 2]
            if co < cw:
                src = src.at[:, pl.ds(0, co)]
            return pltpu.async_copy(
                src, out_hbm.at[pl.ds(base + c * CW, CW)], osems[c % 2])

        g = gather(0)
        prev_w = None
        for c in range(NCHUNK):
            g.wait()
            w = put(c)
            if prev_w is not None:
                prev_w.wait()
            if c + 1 < NCHUNK:
                g = gather(c + 1)
            prev_w = w
        prev_w.wait()

    return gsc(table_flat, idx_flat)


def _edge_body(c, cout, cpo, feats_ref, hprev_ref, waT_ref, wbT_ref, b_ref,
               out_ref):
    feat = feats_ref[0][:, :c]
    center = hprev_ref[0][:, :c]
    feat3 = feat.reshape(NP, K, c)
    ea = (feat3 - center[:, None, :]).astype(jnp.bfloat16)
    za = lax.dot_general(ea.reshape(NP * K, c), waT_ref[...],
                         (((1,), (0,)), ((), ())),
                         preferred_element_type=jnp.float32)
    cb = center.astype(jnp.bfloat16)
    zb = lax.dot_general(cb, wbT_ref[...], (((1,), (0,)), ((), ())),
                         preferred_element_type=jnp.float32)
    z = za.reshape(NP, K, cout) + zb[:, None, :]
    z = z + b_ref[...][None]
    z = jnp.where(z >= 0, z, 0.2 * z)
    hout = jnp.max(z, axis=1)
    if cpo > cout:
        hout = jnp.concatenate(
            [hout, jnp.zeros((NP, cpo - cout), jnp.float32)], axis=1)
    out_ref[0] = hout


def _run_edge(feats, h_prev, waT_bf, wbT_bf, bias, cpo):
    c = waT_bf.shape[0]
    cout = waT_bf.shape[1]
    nc = N // NP
    nb = feats.shape[0]
    full = lambda b, i: (0, 0)
    return pl.pallas_call(
        functools.partial(_edge_body, c, cout, cpo),
        grid=(nb, nc),
        in_specs=[
            pl.BlockSpec((1, NP * K, feats.shape[2]), lambda b, i: (b, i, 0)),
            pl.BlockSpec((1, NP, h_prev.shape[2]), lambda b, i: (b, i, 0)),
            pl.BlockSpec((c, cout), full),
            pl.BlockSpec((c, cout), full),
            pl.BlockSpec((1, cout), full),
        ],
        out_specs=pl.BlockSpec((1, NP, cpo), lambda b, i: (b, i, 0)),
        out_shape=jax.ShapeDtypeStruct((nb, N, cpo), jnp.float32),
    )(feats, h_prev, waT_bf, wbT_bf, bias.reshape(1, cout))


def _final_body(h1_ref, h2_ref, h3_ref, h4_ref,
                w1_ref, w2_ref, w3_ref, w4_ref, b_ref, out_ref):
    acc = lax.dot_general(h1_ref[0][:, :64].astype(jnp.bfloat16), w1_ref[...],
                          (((1,), (0,)), ((), ())),
                          preferred_element_type=jnp.float32)
    acc += lax.dot_general(h2_ref[0][:, :64].astype(jnp.bfloat16), w2_ref[...],
                           (((1,), (0,)), ((), ())),
                           preferred_element_type=jnp.float32)
    acc += lax.dot_general(h3_ref[0].astype(jnp.bfloat16), w3_ref[...],
                           (((1,), (0,)), ((), ())),
                           preferred_element_type=jnp.float32)
    acc += lax.dot_general(h4_ref[0].astype(jnp.bfloat16), w4_ref[...],
                           (((1,), (0,)), ((), ())),
                           preferred_element_type=jnp.float32)
    out_ref[0] = jnp.max(acc, axis=0)[None, :] + b_ref[...]


def _run_final(h1, h2, h3, h4, wf_parts, bf):
    fdim = bf.shape[0]
    nb = h1.shape[0]
    full = lambda b: (0, 0)
    bat = lambda b: (b, 0, 0)
    in_specs = [
        pl.BlockSpec((1, N, h1.shape[2]), bat),
        pl.BlockSpec((1, N, h2.shape[2]), bat),
        pl.BlockSpec((1, N, h3.shape[2]), bat),
        pl.BlockSpec((1, N, h4.shape[2]), bat),
    ] + [pl.BlockSpec(w.shape, full) for w in wf_parts] \
      + [pl.BlockSpec((1, fdim), full)]
    return pl.pallas_call(
        _final_body,
        grid=(nb,),
        in_specs=in_specs,
        out_specs=pl.BlockSpec((1, 1, fdim), lambda b: (b, 0, 0)),
        out_shape=jax.ShapeDtypeStruct((nb, 1, fdim), jnp.float32),
    )(h1, h2, h3, h4, *wf_parts, bf.reshape(1, fdim)).reshape(nb, fdim)


NGROUP = 2


def _group_chain(x, Ws, bs, wf_parts, bf):
    cins = (3, 64, 64, 128)
    pads = (128, 128, 128, 256)
    nb = x.shape[0]
    h_pad = jnp.pad(x, ((0, 0), (0, 0), (0, CP - x.shape[2])))
    acts = []
    for i in range(4):
        c = cins[i]
        waT = jnp.transpose(Ws[i][:, :c]).astype(jnp.bfloat16)
        wbT = jnp.transpose(Ws[i][:, c:]).astype(jnp.bfloat16)
        idx = _run_dist(h_pad, c)
        feats = _gather_sc(h_pad.reshape(nb * N, CP),
                           idx.reshape(nb * N * K), CP)
        feats = feats.reshape(nb, N * K, CP)
        h_next = _run_edge(feats, h_pad, waT, wbT, bs[i], pads[i])
        acts.append(h_next)
        h_pad = h_next
    return _run_final(acts[0], acts[1], acts[2], acts[3], wf_parts, bf)


def kernel(x, W0, b0, W1, b1, W2, b2, W3, b3, Wf, bf):
    Ws = (W0, W1, W2, W3)
    bs = (b0, b1, b2, b3)
    wfT = jnp.transpose(Wf).astype(jnp.bfloat16)
    wf_parts = [wfT[0:64], wfT[64:128], wfT[128:256], wfT[256:512]]
    gb = B // NGROUP
    outs = [_group_chain(x[g * gb:(g + 1) * gb], Ws, bs, wf_parts, bf)
            for g in range(NGROUP)]
    return jnp.concatenate(outs, axis=0)
